# Initial kernel scaffold; baseline (speedup 1.0000x reference)
import jax
import jax.numpy as jnp
from jax.experimental import pallas as pl


def kernel(x, edge_index):
    raise NotImplementedError("write your pallas kernel here")


# speedup vs baseline: 5.8809x; 5.8809x over previous
import functools

import jax
import jax.numpy as jnp
from jax import lax
from jax.experimental import pallas as pl
from jax.experimental.pallas import tpu as pltpu
from jax.experimental.pallas import tpu_sc as plsc

N_NODES = 10000
D_FEAT = 128
HALF = 64
N_EDGES = 320000
NC = 2
NS = 16
CHUNK = 128
CH_PER_TILE = 157
E_TILE = CHUNK * CH_PER_TILE
E_PAD = E_TILE * NS
ROWS_PAD = 10112
ROWS_TILE = ROWS_PAD // NS
RCHUNK = 632
NRCH = ROWS_TILE // RCHUNK
WEIGHT = 0.1


def _sc_agg(xh, src2, dst3):
    mesh = plsc.VectorSubcoreMesh(core_axis_name="c", subcore_axis_name="s")

    @functools.partial(
        pl.kernel,
        out_type=jax.ShapeDtypeStruct((NC, ROWS_PAD, HALF), jnp.float32),
        mesh=mesh,
        scratch_types=[
            pltpu.VMEM((CH_PER_TILE, CHUNK), jnp.int32),
            pltpu.VMEM((CH_PER_TILE, CHUNK), jnp.int32),
            pltpu.VMEM((CHUNK, HALF), jnp.float32),
            pltpu.VMEM((RCHUNK, HALF), jnp.float32),
            pltpu.VMEM_SHARED((ROWS_PAD, HALF), jnp.float32),
            pltpu.SemaphoreType.DMA,
        ],
        compiler_params=pltpu.CompilerParams(use_tc_tiling_on_sc=False),
    )
    def k(xh_hbm, src_hbm, dst_hbm, out_hbm, src_v, dst_v, rows_v, xbuf, acc, sem):
        c = lax.axis_index("c")
        s = lax.axis_index("s")
        row0 = s * ROWS_TILE
        xrow0 = c * ROWS_PAD

        pltpu.sync_copy(src_hbm.at[c, s], src_v)
        pltpu.sync_copy(dst_hbm.at[s], dst_v)

        def init_chunk(r, _):
            base = row0 + r * RCHUNK
            pltpu.sync_copy(xh_hbm.at[pl.ds(xrow0 + base, RCHUNK)], xbuf)

            def scale_row(i, _):
                for j in range(HALF // 16):
                    sl = pl.ds(j * 16, 16)
                    xbuf[i, sl] = xbuf[i, sl] * 10.0
                return 0

            lax.fori_loop(0, RCHUNK, scale_row, 0)
            pltpu.sync_copy(xbuf, acc.at[pl.ds(base, RCHUNK)])
            return 0

        lax.fori_loop(0, NRCH, init_chunk, 0)
        plsc.subcore_barrier()

        def edge_chunk(j, _):
            pltpu.async_copy(xh_hbm.at[src_v.at[j]], rows_v, sem).wait()
            pltpu.sync_copy(rows_v, acc.at[dst_v.at[j]], add=True)
            return 0

        lax.fori_loop(0, CH_PER_TILE, edge_chunk, 0)
        plsc.subcore_barrier()

        def fin_chunk(r, _):
            base = row0 + r * RCHUNK
            pltpu.sync_copy(acc.at[pl.ds(base, RCHUNK)], xbuf)

            def scale_row(i, _):
                for j in range(HALF // 16):
                    sl = pl.ds(j * 16, 16)
                    xbuf[i, sl] = xbuf[i, sl] * jnp.float32(WEIGHT)
                return 0

            lax.fori_loop(0, RCHUNK, scale_row, 0)
            pltpu.sync_copy(xbuf, out_hbm.at[c, pl.ds(base, RCHUNK)])
            return 0

        lax.fori_loop(0, NRCH, fin_chunk, 0)

    return k(xh, src2, dst3)


@jax.jit
def kernel(x, edge_index):
    src = edge_index[0]
    dst = edge_index[1]
    pad_rows = ROWS_PAD - N_NODES
    xh = jnp.concatenate(
        [
            jnp.pad(x[:, :HALF], ((0, pad_rows), (0, 0))),
            jnp.pad(x[:, HALF:], ((0, pad_rows), (0, 0))),
        ],
        axis=0,
    )
    srcp = jnp.pad(src, (0, E_PAD - N_EDGES))
    dstp = jnp.pad(dst, (0, E_PAD - N_EDGES), constant_values=N_NODES)
    src2 = jnp.stack([srcp, srcp + ROWS_PAD]).reshape(NC, NS, CH_PER_TILE, CHUNK)
    dst3 = dstp.reshape(NS, CH_PER_TILE, CHUNK)
    o = _sc_agg(xh, src2, dst3)
    return jnp.concatenate([o[0, :N_NODES], o[1, :N_NODES]], axis=1)
